# baseline (speedup 1.0000x reference)
import jax
import jax.numpy as jnp
from jax.experimental import pallas as pl
from jax.experimental.pallas import tpu as pltpu


def _probe_body(h_ref, o_ref):
    i = pl.program_id(0)
    dv = jnp.sum(h_ref[...], axis=1, keepdims=True)

    @pl.when(i == 0)
    def _():
        o_ref[...] = jnp.zeros_like(o_ref)

    o_ref[...] += jnp.sum(dv)


def kernel(x_0, x_1, incidence_1, params):
    hpad = jnp.pad(incidence_1, ((0, 0), (0, 60)))
    BN = 1000
    NB = 10
    s = pl.pallas_call(
        _probe_body,
        grid=(2 * NB,),
        in_specs=[pl.BlockSpec((BN, hpad.shape[1]),
                               lambda i: (jax.lax.rem(i, NB), 0))],
        out_specs=pl.BlockSpec((1, 1), lambda i: (0, 0)),
        out_shape=jax.ShapeDtypeStruct((1, 1), jnp.float32),
        compiler_params=pltpu.CompilerParams(
            dimension_semantics=("arbitrary",)),
    )(hpad)
    return x_0 + s, x_1 + s


# speedup vs baseline: 3.0973x; 3.0973x over previous
import jax
import jax.numpy as jnp
from jax.experimental import pallas as pl
from jax.experimental.pallas import tpu as pltpu

_NB = 10
_MP = 2560


def _ln(x, g, b):
    mu = jnp.mean(x, axis=-1, keepdims=True)
    var = jnp.mean((x - mu) ** 2, axis=-1, keepdims=True)
    return g * (x - mu) * jax.lax.rsqrt(var + 1e-5) + b


def _p1_body(h_ref, x0_ref, x1_ref, whe_ref, bhe_ref, wv_ref,
             hb_ref, rv_ref, x1new_ref, x1vp_ref, rep_ref,
             acc_ref, de_ref):
    i = pl.program_id(0)
    h = h_ref[...]
    dv = jnp.sum(h, axis=1, keepdims=True)
    rv = jax.lax.rsqrt(jnp.maximum(dv, 1.0))
    rv_ref[...] = rv
    hb = h.astype(jnp.bfloat16)
    hb_ref[...] = jnp.concatenate(
        [hb, jnp.zeros((h.shape[0], _MP - h.shape[1]), jnp.bfloat16)],
        axis=1)
    x0s = (x0_ref[...] * rv).astype(jnp.bfloat16)
    contrib = jax.lax.dot_general(
        x0s, hb, (((0,), (0,)), ((), ())),
        preferred_element_type=jnp.float32)
    de_c = jnp.sum(h, axis=0, keepdims=True)

    @pl.when(i == 0)
    def _():
        acc_ref[...] = contrib
        de_ref[...] = de_c

    @pl.when(i != 0)
    def _():
        acc_ref[...] += contrib
        de_ref[...] += de_c

    @pl.when(i == _NB - 1)
    def _epilogue():
        re = jax.lax.rsqrt(jnp.maximum(de_ref[...], 1.0))
        rep_ref[...] = jnp.concatenate(
            [re, jnp.ones((1, _MP - re.shape[1]), jnp.float32)], axis=1)
        accs = acc_ref[...] * re
        msg = jax.lax.dot_general(
            accs, whe_ref[...], (((0,), (0,)), ((), ())),
            preferred_element_type=jnp.float32)
        x1new = x1_ref[...] + msg + bhe_ref[...]
        x1new_ref[...] = x1new
        re_col = jnp.transpose(re)
        x1v = jnp.dot(x1new * re_col, wv_ref[...],
                      preferred_element_type=jnp.float32).astype(jnp.bfloat16)
        x1vp_ref[...] = jnp.concatenate(
            [x1v, jnp.zeros((_MP - x1v.shape[0], x1v.shape[1]),
                            jnp.bfloat16)], axis=0)


def _p2_body(hb_ref, x0_ref, rv_ref, x1vp_ref, rep_ref, x1new_ref,
             bv_ref, tgl_ref, tgr_ref, n1g_ref, n1b_ref, n2g_ref, n2b_ref,
             w1_ref, b1_ref, w2_ref, b2_ref, wret_ref, bret_ref,
             xout_ref, x1outp_ref, ret_ref):
    i = pl.program_id(0)
    hb = hb_ref[...]
    rv = rv_ref[...]
    msgv = jax.lax.dot_general(
        hb, x1vp_ref[...], (((1,), (0,)), ((), ())),
        preferred_element_type=jnp.float32) * rv
    t = x0_ref[...] + tgl_ref[...] * (msgv + bv_ref[...])
    x0l = _ln(t, n1g_ref[...], n1b_ref[...])
    x0g = _ln(x0l, n2g_ref[...], n2b_ref[...])
    pre = jax.lax.dot_general(
        x0g.astype(jnp.bfloat16), w1_ref[...], (((1,), (0,)), ((), ())),
        preferred_element_type=jnp.float32) + b1_ref[...]
    hmid = pre * 0.5 * (1.0 + jax.lax.erf(pre * 0.7071067811865476))
    xout_ref[...] = x0g + jax.lax.dot_general(
        hmid.astype(jnp.bfloat16), w2_ref[...], (((1,), (0,)), ((), ())),
        preferred_element_type=jnp.float32) + b2_ref[...]
    x0ls = (x0l * rv).astype(jnp.bfloat16)
    contrib = jax.lax.dot_general(
        x0ls, hb, (((0,), (0,)), ((), ())),
        preferred_element_type=jnp.float32)

    @pl.when(i == 0)
    def _():
        ret_ref[...] = contrib

    @pl.when(i != 0)
    def _():
        ret_ref[...] += contrib

    @pl.when(i == _NB - 1)
    def _epilogue():
        rets = ret_ref[...] * rep_ref[...]
        msg = jax.lax.dot_general(
            rets, wret_ref[...], (((0,), (0,)), ((), ())),
            preferred_element_type=jnp.float32)
        x1new = x1new_ref[...]
        x1pad = jnp.concatenate(
            [x1new, jnp.zeros((_MP - x1new.shape[0], x1new.shape[1]),
                              jnp.float32)], axis=0)
        x1outp_ref[...] = x1pad + tgr_ref[...] * (msg + bret_ref[...])


def kernel(x_0, x_1, incidence_1, params):
    N, D = x_0.shape
    M = x_1.shape[0]
    p = params
    f32 = jnp.float32
    bf16 = jnp.bfloat16
    BN = N // _NB

    tgl = jnp.tanh(p["gate_local"]).reshape(1, 1)
    tgr = jnp.tanh(p["gate_return"]).reshape(1, 1)

    tile = lambda w: pl.BlockSpec((BN, w), lambda i: (i, 0))
    const = lambda shape: pl.BlockSpec(shape, lambda i: (0,) * len(shape))

    hb_pad, rv_all, x1new, x1vp, re_pad = pl.pallas_call(
        _p1_body,
        grid=(_NB,),
        in_specs=[
            tile(M),
            tile(D),
            const((M, D)),
            const((D, D)),
            const((1, D)),
            const((D, D)),
        ],
        out_specs=[
            tile(_MP),
            pl.BlockSpec((BN, 1), lambda i: (i, 0)),
            const((M, D)),
            const((_MP, D)),
            const((1, _MP)),
        ],
        out_shape=[
            jax.ShapeDtypeStruct((N, _MP), bf16),
            jax.ShapeDtypeStruct((N, 1), f32),
            jax.ShapeDtypeStruct((M, D), f32),
            jax.ShapeDtypeStruct((_MP, D), bf16),
            jax.ShapeDtypeStruct((1, _MP), f32),
        ],
        scratch_shapes=[
            pltpu.VMEM((D, M), f32),
            pltpu.VMEM((1, M), f32),
        ],
        compiler_params=pltpu.CompilerParams(
            dimension_semantics=("arbitrary",)),
    )(incidence_1, x_0, x_1, p["W_he"], p["b_he"].reshape(1, D), p["W_v"])

    x_out, x1out_pad = pl.pallas_call(
        _p2_body,
        grid=(_NB,),
        in_specs=[
            tile(_MP),
            tile(D),
            pl.BlockSpec((BN, 1), lambda i: (i, 0)),
            const((_MP, D)),
            const((1, _MP)),
            const((M, D)),
            const((1, D)),
            const((1, 1)),
            const((1, 1)),
            const((1, D)),
            const((1, D)),
            const((1, D)),
            const((1, D)),
            const((D, 2 * D)),
            const((1, 2 * D)),
            const((2 * D, D)),
            const((1, D)),
            const((D, D)),
            const((1, D)),
        ],
        out_specs=[
            tile(D),
            const((_MP, D)),
        ],
        out_shape=[
            jax.ShapeDtypeStruct((N, D), f32),
            jax.ShapeDtypeStruct((_MP, D), f32),
        ],
        scratch_shapes=[
            pltpu.VMEM((D, _MP), f32),
        ],
        compiler_params=pltpu.CompilerParams(
            dimension_semantics=("arbitrary",)),
    )(hb_pad, x_0, rv_all, x1vp, re_pad, x1new,
      p["b_v"].reshape(1, D), tgl, tgr,
      p["n1_g"].reshape(1, D), p["n1_b"].reshape(1, D),
      p["n2_g"].reshape(1, D), p["n2_b"].reshape(1, D),
      p["W1"].astype(bf16), p["b1"].reshape(1, 2 * D),
      p["W2"].astype(bf16), p["b2"].reshape(1, D),
      p["W_ret"], p["b_ret"].reshape(1, D))

    return x_out, x1out_pad[:M]


# speedup vs baseline: 3.3399x; 1.0783x over previous
import jax
import jax.numpy as jnp
from jax.experimental import pallas as pl
from jax.experimental.pallas import tpu as pltpu

_NB = 10
_NCACHE = 4


def _ln(x, g, b):
    mu = jnp.mean(x, axis=-1, keepdims=True)
    var = jnp.mean((x - mu) ** 2, axis=-1, keepdims=True)
    return g * (x - mu) * jax.lax.rsqrt(var + 1e-5) + b


def _mega_body(h_ref, x0_ref, x1_ref, whe_ref, bhe_ref, wv_ref, bv_ref,
               tgl_ref, tgr_ref, n1g_ref, n1b_ref, n2g_ref, n2b_ref,
               w1_ref, b1_ref, w2_ref, b2_ref, wret_ref, bret_ref,
               xout_ref, x1out_ref,
               cache_ref, acc_ref, de_ref, x1new_ref, x1v_ref,
               re_ref, ret_ref):
    i = pl.program_id(0)

    @pl.when(i < _NB)
    def _phase1():
        h = h_ref[...]
        dv = jnp.sum(h, axis=1, keepdims=True)
        rv = jax.lax.rsqrt(jnp.maximum(dv, 1.0))
        hb = h.astype(jnp.bfloat16)

        @pl.when(i < _NCACHE)
        def _():
            cache_ref[i] = hb

        x0s = (x0_ref[...] * rv).astype(jnp.bfloat16)
        contrib = jax.lax.dot_general(
            x0s, hb, (((0,), (0,)), ((), ())),
            preferred_element_type=jnp.float32)
        de_c = jnp.sum(h, axis=0, keepdims=True)

        @pl.when(i == 0)
        def _():
            acc_ref[...] = contrib
            de_ref[...] = de_c

        @pl.when(i != 0)
        def _():
            acc_ref[...] += contrib
            de_ref[...] += de_c

        @pl.when(i == _NB - 1)
        def _k1_epilogue():
            re = jax.lax.rsqrt(jnp.maximum(de_ref[...], 1.0))
            re_ref[...] = re
            accs = acc_ref[...] * re
            msg = jax.lax.dot_general(
                accs, whe_ref[...], (((0,), (0,)), ((), ())),
                preferred_element_type=jnp.float32)
            x1new = x1_ref[...] + msg + bhe_ref[...]
            x1new_ref[...] = x1new
            re_col = jnp.transpose(re)
            x1v_ref[...] = jnp.dot(x1new * re_col, wv_ref[...],
                                   preferred_element_type=jnp.float32
                                   ).astype(jnp.bfloat16)

    def _phase2_tile(hb, j):
        dv = jnp.sum(hb.astype(jnp.float32), axis=1, keepdims=True)
        rv = jax.lax.rsqrt(jnp.maximum(dv, 1.0))
        msgv = jax.lax.dot_general(
            hb, x1v_ref[...], (((1,), (0,)), ((), ())),
            preferred_element_type=jnp.float32) * rv
        t = x0_ref[...] + tgl_ref[...] * (msgv + bv_ref[...])
        x0l = _ln(t, n1g_ref[...], n1b_ref[...])
        x0g = _ln(x0l, n2g_ref[...], n2b_ref[...])
        pre = jax.lax.dot_general(
            x0g.astype(jnp.bfloat16), w1_ref[...], (((1,), (0,)), ((), ())),
            preferred_element_type=jnp.float32) + b1_ref[...]
        hmid = pre * 0.5 * (1.0 + jax.lax.erf(pre * 0.7071067811865476))
        xout_ref[...] = x0g + jax.lax.dot_general(
            hmid.astype(jnp.bfloat16), w2_ref[...], (((1,), (0,)), ((), ())),
            preferred_element_type=jnp.float32) + b2_ref[...]
        x0ls = (x0l * rv).astype(jnp.bfloat16)
        contrib = jax.lax.dot_general(
            x0ls, hb, (((0,), (0,)), ((), ())),
            preferred_element_type=jnp.float32)

        @pl.when(j == 0)
        def _():
            ret_ref[...] = contrib

        @pl.when(j != 0)
        def _():
            ret_ref[...] += contrib

        @pl.when(j == _NB - 1)
        def _k2_epilogue():
            rets = ret_ref[...] * re_ref[...]
            msg = jax.lax.dot_general(
                rets, wret_ref[...], (((0,), (0,)), ((), ())),
                preferred_element_type=jnp.float32)
            x1out_ref[...] = x1new_ref[...] + tgr_ref[...] * (
                msg + bret_ref[...])

    @pl.when((i >= _NB) & (i < _NB + _NCACHE))
    def _phase2_cached():
        _phase2_tile(cache_ref[i - _NB], i - _NB)

    @pl.when(i >= _NB + _NCACHE)
    def _phase2_streamed():
        _phase2_tile(h_ref[...].astype(jnp.bfloat16), i - _NB)


def kernel(x_0, x_1, incidence_1, params):
    N, D = x_0.shape
    M = x_1.shape[0]
    p = params
    f32 = jnp.float32
    bf16 = jnp.bfloat16
    BN = N // _NB
    nb, nc = _NB, _NCACHE

    tgl = jnp.tanh(p["gate_local"]).reshape(1, 1)
    tgr = jnp.tanh(p["gate_return"]).reshape(1, 1)

    def h_idx(i):
        return (jnp.where(i < nb, i, jnp.where(i < nb + nc, nb - 1, i - nb)),
                0)

    const = lambda shape: pl.BlockSpec(shape, lambda i: (0,) * len(shape))

    x_out, x1out = pl.pallas_call(
        _mega_body,
        grid=(2 * _NB,),
        in_specs=[
            pl.BlockSpec((BN, M), h_idx),
            pl.BlockSpec((BN, D), lambda i: (jax.lax.rem(i, nb), 0)),
            const((M, D)),
            const((D, D)),
            const((1, D)),
            const((D, D)),
            const((1, D)),
            const((1, 1)),
            const((1, 1)),
            const((1, D)),
            const((1, D)),
            const((1, D)),
            const((1, D)),
            const((D, 2 * D)),
            const((1, 2 * D)),
            const((2 * D, D)),
            const((1, D)),
            const((D, D)),
            const((1, D)),
        ],
        out_specs=[
            pl.BlockSpec(
                (BN, D),
                lambda i: (jnp.where(i < nb, 0, i - nb), 0)),
            const((M, D)),
        ],
        out_shape=[
            jax.ShapeDtypeStruct((N, D), f32),
            jax.ShapeDtypeStruct((M, D), f32),
        ],
        scratch_shapes=[
            pltpu.VMEM((_NCACHE, BN, M), bf16),
            pltpu.VMEM((D, M), f32),
            pltpu.VMEM((1, M), f32),
            pltpu.VMEM((M, D), f32),
            pltpu.VMEM((M, D), bf16),
            pltpu.VMEM((1, M), f32),
            pltpu.VMEM((D, M), f32),
        ],
        compiler_params=pltpu.CompilerParams(
            dimension_semantics=("arbitrary",),
            vmem_limit_bytes=67108864,
        ),
    )(incidence_1, x_0, x_1,
      p["W_he"], p["b_he"].reshape(1, D), p["W_v"], p["b_v"].reshape(1, D),
      tgl, tgr,
      p["n1_g"].reshape(1, D), p["n1_b"].reshape(1, D),
      p["n2_g"].reshape(1, D), p["n2_b"].reshape(1, D),
      p["W1"].astype(bf16), p["b1"].reshape(1, 2 * D),
      p["W2"].astype(bf16), p["b2"].reshape(1, D),
      p["W_ret"], p["b_ret"].reshape(1, D))

    return x_out, x1out


# speedup vs baseline: 3.3829x; 1.0129x over previous
import jax
import jax.numpy as jnp
from jax.experimental import pallas as pl
from jax.experimental.pallas import tpu as pltpu

_NB = 10
_NCACHE = 4


def _ln(x, g, b):
    mu = jnp.mean(x, axis=-1, keepdims=True)
    var = jnp.mean((x - mu) ** 2, axis=-1, keepdims=True)
    return g * (x - mu) * jax.lax.rsqrt(var + 1e-5) + b


def _mega_body(h_ref, x0_ref, x1_ref, whe_ref, bhe_ref, wv_ref, bv_ref,
               tgl_ref, tgr_ref, n1g_ref, n1b_ref, n2g_ref, n2b_ref,
               w1_ref, b1_ref, w2_ref, b2_ref, wret_ref, bret_ref,
               xout_ref, x1out_ref,
               cache_ref, acc_ref, de_ref, x1new_ref, x1v_ref,
               re_ref, ret_ref):
    i = pl.program_id(0)

    @pl.when(i < _NB)
    def _phase1():
        h = h_ref[...]
        dv = jnp.sum(h, axis=1, keepdims=True)
        rv = jax.lax.rsqrt(jnp.maximum(dv, 1.0))
        hb = h.astype(jnp.bfloat16)

        @pl.when(i < _NCACHE)
        def _():
            cache_ref[i] = (h * rv).astype(jnp.bfloat16)

        x0s = (x0_ref[...] * rv).astype(jnp.bfloat16)
        contrib = jax.lax.dot_general(
            x0s, hb, (((0,), (0,)), ((), ())),
            preferred_element_type=jnp.float32)
        de_c = jnp.sum(h, axis=0, keepdims=True)

        @pl.when(i == 0)
        def _():
            acc_ref[...] = contrib
            de_ref[...] = de_c

        @pl.when(i != 0)
        def _():
            acc_ref[...] += contrib
            de_ref[...] += de_c

        @pl.when(i == _NB - 1)
        def _k1_epilogue():
            re = jax.lax.rsqrt(jnp.maximum(de_ref[...], 1.0))
            re_ref[...] = re
            accs = acc_ref[...] * re
            msg = jax.lax.dot_general(
                accs, whe_ref[...], (((0,), (0,)), ((), ())),
                preferred_element_type=jnp.float32)
            x1new = x1_ref[...] + msg + bhe_ref[...]
            x1new_ref[...] = x1new
            re_col = jnp.transpose(re)
            x1v_ref[...] = jnp.dot(x1new * re_col, wv_ref[...],
                                   preferred_element_type=jnp.float32
                                   ).astype(jnp.bfloat16)

    def _phase2_tile(hbs, j):
        msgv = jax.lax.dot_general(
            hbs, x1v_ref[...], (((1,), (0,)), ((), ())),
            preferred_element_type=jnp.float32)
        t = x0_ref[...] + tgl_ref[...] * (msgv + bv_ref[...])
        x0l = _ln(t, n1g_ref[...], n1b_ref[...])
        x0g = _ln(x0l, n2g_ref[...], n2b_ref[...])
        pre = jax.lax.dot_general(
            x0g.astype(jnp.bfloat16), w1_ref[...], (((1,), (0,)), ((), ())),
            preferred_element_type=jnp.float32) + b1_ref[...]
        hmid = pre * 0.5 * (1.0 + jax.lax.erf(pre * 0.7071067811865476))
        xout_ref[...] = x0g + jax.lax.dot_general(
            hmid.astype(jnp.bfloat16), w2_ref[...], (((1,), (0,)), ((), ())),
            preferred_element_type=jnp.float32) + b2_ref[...]
        x0ls = x0l.astype(jnp.bfloat16)
        contrib = jax.lax.dot_general(
            x0ls, hbs, (((0,), (0,)), ((), ())),
            preferred_element_type=jnp.float32)

        @pl.when(j == 0)
        def _():
            ret_ref[...] = contrib

        @pl.when(j != 0)
        def _():
            ret_ref[...] += contrib

        @pl.when(j == _NB - 1)
        def _k2_epilogue():
            rets = ret_ref[...] * re_ref[...]
            msg = jax.lax.dot_general(
                rets, wret_ref[...], (((0,), (0,)), ((), ())),
                preferred_element_type=jnp.float32)
            x1out_ref[...] = x1new_ref[...] + tgr_ref[...] * (
                msg + bret_ref[...])

    @pl.when((i >= _NB) & (i < _NB + _NCACHE))
    def _phase2_cached():
        _phase2_tile(cache_ref[i - _NB], i - _NB)

    @pl.when(i >= _NB + _NCACHE)
    def _phase2_streamed():
        h = h_ref[...]
        dv = jnp.sum(h, axis=1, keepdims=True)
        rv = jax.lax.rsqrt(jnp.maximum(dv, 1.0))
        _phase2_tile((h * rv).astype(jnp.bfloat16), i - _NB)


def kernel(x_0, x_1, incidence_1, params):
    N, D = x_0.shape
    M = x_1.shape[0]
    p = params
    f32 = jnp.float32
    bf16 = jnp.bfloat16
    BN = N // _NB
    nb, nc = _NB, _NCACHE

    tgl = jnp.tanh(p["gate_local"]).reshape(1, 1)
    tgr = jnp.tanh(p["gate_return"]).reshape(1, 1)

    def h_idx(i):
        return (jnp.where(i < nb, i, jnp.where(i < nb + nc, nb - 1, i - nb)),
                0)

    const = lambda shape: pl.BlockSpec(shape, lambda i: (0,) * len(shape))

    x_out, x1out = pl.pallas_call(
        _mega_body,
        grid=(2 * _NB,),
        in_specs=[
            pl.BlockSpec((BN, M), h_idx),
            pl.BlockSpec((BN, D), lambda i: (jax.lax.rem(i, nb), 0)),
            const((M, D)),
            const((D, D)),
            const((1, D)),
            const((D, D)),
            const((1, D)),
            const((1, 1)),
            const((1, 1)),
            const((1, D)),
            const((1, D)),
            const((1, D)),
            const((1, D)),
            const((D, 2 * D)),
            const((1, 2 * D)),
            const((2 * D, D)),
            const((1, D)),
            const((D, D)),
            const((1, D)),
        ],
        out_specs=[
            pl.BlockSpec(
                (BN, D),
                lambda i: (jnp.where(i < nb, 0, i - nb), 0)),
            const((M, D)),
        ],
        out_shape=[
            jax.ShapeDtypeStruct((N, D), f32),
            jax.ShapeDtypeStruct((M, D), f32),
        ],
        scratch_shapes=[
            pltpu.VMEM((_NCACHE, BN, M), bf16),
            pltpu.VMEM((D, M), f32),
            pltpu.VMEM((1, M), f32),
            pltpu.VMEM((M, D), f32),
            pltpu.VMEM((M, D), bf16),
            pltpu.VMEM((1, M), f32),
            pltpu.VMEM((D, M), f32),
        ],
        compiler_params=pltpu.CompilerParams(
            dimension_semantics=("arbitrary",),
            vmem_limit_bytes=67108864,
        ),
    )(incidence_1, x_0, x_1,
      p["W_he"], p["b_he"].reshape(1, D), p["W_v"], p["b_v"].reshape(1, D),
      tgl, tgr,
      p["n1_g"].reshape(1, D), p["n1_b"].reshape(1, D),
      p["n2_g"].reshape(1, D), p["n2_b"].reshape(1, D),
      p["W1"].astype(bf16), p["b1"].reshape(1, 2 * D),
      p["W2"].astype(bf16), p["b2"].reshape(1, D),
      p["W_ret"], p["b_ret"].reshape(1, D))

    return x_out, x1out
